# Initial kernel scaffold; baseline (speedup 1.0000x reference)
import jax
import jax.numpy as jnp
from jax.experimental import pallas as pl


def kernel(cg_z, cg_xyz, CG_nbr_list, mapping, S, params):
    raise NotImplementedError("write your pallas kernel here")


# speedup vs baseline: 2.6031x; 2.6031x over previous
import functools

import jax
import jax.numpy as jnp
import numpy as np
from jax import lax
from jax.experimental import pallas as pl
from jax.experimental.pallas import tpu as pltpu
from jax.experimental.pallas import tpu_sc as plsc

N_ATOM_BASIS = 128
RES_DIM = 4
FEAT = N_ATOM_BASIS + RES_DIM
FP = 160
HF = FP // 2
N_RBF = 15
CUTOFF = 21.0
NUM_CONV = 3

NSUB = 16
NCORE = 2
NW = NCORE * NSUB

_SC_PARAMS = pltpu.CompilerParams(needs_layout_passes=False,
                                  use_tc_tiling_on_sc=False)


def _silu(x):
    return x * jax.nn.sigmoid(x)


def _pad2(w, rows, cols):
    return jnp.zeros((rows, cols), w.dtype).at[: w.shape[0], : w.shape[1]].set(w)


def _pad1(b, n):
    return jnp.zeros((n,), b.dtype).at[: b.shape[0]].set(b)


def _sc_dist(n_nodes, n_edges):
    m = n_edges // NW
    assert n_edges % NW == 0 and m % 16 == 0

    mesh = plsc.VectorSubcoreMesh(core_axis_name="c", subcore_axis_name="s",
                                  num_cores=NCORE, num_subcores=NSUB)

    @functools.partial(
        pl.kernel,
        out_type=jax.ShapeDtypeStruct((n_edges,), jnp.float32),
        mesh=mesh,
        scratch_types=[
            pltpu.VMEM((n_nodes,), jnp.float32),
            pltpu.VMEM((n_nodes,), jnp.float32),
            pltpu.VMEM((n_nodes,), jnp.float32),
            pltpu.VMEM((m,), jnp.int32),
            pltpu.VMEM((m,), jnp.int32),
            pltpu.VMEM((m,), jnp.float32),
        ],
        compiler_params=_SC_PARAMS,
    )
    def k(x_hbm, y_hbm, z_hbm, src_hbm, dst_hbm, out_hbm, xv, yv, zv,
          sv, dv, ov):
        cid = lax.axis_index("c")
        sid = lax.axis_index("s")
        wid = cid * NSUB + sid
        base = wid * m
        pltpu.sync_copy(x_hbm, xv)
        pltpu.sync_copy(y_hbm, yv)
        pltpu.sync_copy(z_hbm, zv)
        pltpu.sync_copy(src_hbm.at[pl.ds(base, m)], sv)
        pltpu.sync_copy(dst_hbm.at[pl.ds(base, m)], dv)

        def body(i, _):
            si = sv[pl.ds(i * 16, 16)]
            di = dv[pl.ds(i * 16, 16)]
            dx = plsc.load_gather(xv, [si]) - plsc.load_gather(xv, [di])
            dy = plsc.load_gather(yv, [si]) - plsc.load_gather(yv, [di])
            dz = plsc.load_gather(zv, [si]) - plsc.load_gather(zv, [di])
            ov[pl.ds(i * 16, 16)] = dx * dx + dy * dy + dz * dz
            return ()

        lax.fori_loop(0, m // 16, body, ())
        pltpu.sync_copy(ov, out_hbm.at[pl.ds(base, m)])

    return k


def _sc_conv(n_pad, n_edges):
    m = n_edges // NSUB
    c = 0
    for cand in range(128, 0, -8):
        if m % cand == 0:
            c = cand
            break
    assert c > 0
    j_chunks = m // c
    rows_pt = n_pad // NSUB
    assert n_pad % (NSUB * 8) == 0

    mesh = plsc.VectorSubcoreMesh(core_axis_name="c", subcore_axis_name="s",
                                  num_cores=NCORE, num_subcores=NSUB)

    @functools.partial(
        pl.kernel,
        out_type=[
            jax.ShapeDtypeStruct((n_pad, HF), jnp.float32),
            jax.ShapeDtypeStruct((n_pad, HF), jnp.float32),
        ],
        mesh=mesh,
        scratch_types=[
            pltpu.VMEM((j_chunks, c), jnp.int32),
            pltpu.VMEM((j_chunks, c), jnp.int32),
            pltpu.VMEM((c, HF), jnp.float32),
            pltpu.VMEM((c, HF), jnp.float32),
            pltpu.VMEM_SHARED((n_pad, HF), jnp.float32),
            pltpu.SemaphoreType.DMA,
            pltpu.SemaphoreType.DMA,
        ],
        compiler_params=_SC_PARAMS,
    )
    def k(phia_hbm, phib_hbm, wsa_hbm, wsb_hbm, src_hbm, dst_hbm, zero_hbm,
          out0, out1, sv, dv, pb, wb, acc, sem_g, sem_w):
        cid = lax.axis_index("c")
        sid = lax.axis_index("s")
        pltpu.sync_copy(zero_hbm, acc.at[pl.ds(sid * rows_pt, rows_pt)])
        pltpu.sync_copy(src_hbm.at[sid], sv)
        pltpu.sync_copy(dst_hbm.at[sid], dv)
        plsc.subcore_barrier()

        def run(phi_hbm, ws_hbm, out_hbm):
            def chunk(j, _):
                base = sid * m + j * c
                g = pltpu.async_copy(phi_hbm.at[sv.at[j]], pb, sem_g)
                w = pltpu.async_copy(ws_hbm.at[pl.ds(base, c)], wb, sem_w)
                g.wait()
                w.wait()

                def rowmul(r, _):
                    for q in range(HF // 16):
                        s = pl.ds(q * 16, 16)
                        pb[r, s] = pb[r, s] * wb[r, s]
                    return ()

                lax.fori_loop(0, c, rowmul, ())
                pltpu.sync_copy(pb, acc.at[dv.at[j]], add=True)
                return ()

            lax.fori_loop(0, j_chunks, chunk, ())
            plsc.subcore_barrier()
            row0 = sid * rows_pt
            pltpu.sync_copy(acc.at[pl.ds(row0, rows_pt)],
                            out_hbm.at[pl.ds(row0, rows_pt)])

        @pl.when(cid == 0)
        def _():
            run(phia_hbm, wsa_hbm, out0)

        @pl.when(cid == 1)
        def _():
            run(phib_hbm, wsb_hbm, out1)

    return k


def _dot(a, b):
    return jax.lax.dot_general(a, b, (((1,), (0,)), ((), ())),
                               preferred_element_type=jnp.float32)


def _tc_prep(n_edges, be=4000):
    grid = (n_edges // be,)
    mu0 = float(np.exp(-CUTOFF))
    mu_step = (1.0 - mu0) / (N_RBF - 1)
    beta = float((2.0 / N_RBF * (1.0 - np.exp(-CUTOFF))) ** -2)

    def body(d2_ref, w1_ref, w2_ref, w3_ref, *o_refs):
        d2 = d2_ref[...]
        dist = jnp.sqrt(d2)
        env = jnp.where(dist < CUTOFF,
                        0.5 * (jnp.cos(dist * (np.pi / CUTOFF)) + 1.0),
                        0.0)
        ex = jnp.exp(-dist)
        mu = (mu0 + mu_step * jax.lax.broadcasted_iota(
            jnp.int32, (1, N_RBF), 1).astype(jnp.float32))
        rbf = jnp.exp(-beta * (ex - mu) ** 2)
        rext = jnp.concatenate([rbf, jnp.ones_like(d2)], axis=1) * env
        for i, w_ref in enumerate((w1_ref, w2_ref, w3_ref)):
            ws = _dot(rext, w_ref[...])
            o_refs[2 * i][...] = ws[:, :HF]
            o_refs[2 * i + 1][...] = ws[:, HF:]

    half = jax.ShapeDtypeStruct((n_edges, HF), jnp.float32)
    return pl.pallas_call(
        body,
        grid=grid,
        in_specs=[
            pl.BlockSpec((be, 1), lambda i: (i, 0)),
            pl.BlockSpec((16, FP), lambda i: (0, 0)),
            pl.BlockSpec((16, FP), lambda i: (0, 0)),
            pl.BlockSpec((16, FP), lambda i: (0, 0)),
        ],
        out_specs=[pl.BlockSpec((be, HF), lambda i: (i, 0))] * 6,
        out_shape=[half] * 6,
    )


def _tc_embed(n_nodes, bm=1000):
    grid = (n_nodes // bm,)

    def body(z_ref, s_ref, re_ref, bb_ref, sc_ref, w1_ref, b1_ref,
             w2_ref, b2_ref, sf_ref, phia_ref, phib_ref, bbsc_ref):
        z = z_ref[...]
        onehot = (z == jax.lax.broadcasted_iota(jnp.int32, (1, 25), 1)
                  ).astype(jnp.float32)
        res4 = _dot(onehot, re_ref[...])
        s = s_ref[...]
        pad = jnp.zeros((s.shape[0], FP - FEAT), jnp.float32)
        sfull = jnp.concatenate([s, res4, pad], axis=1)
        sf_ref[...] = sfull
        h = _silu(_dot(sfull, w1_ref[...]) + b1_ref[...])
        phi = _dot(h, w2_ref[...]) + b2_ref[...]
        phia_ref[...] = phi[:, :HF]
        phib_ref[...] = phi[:, HF:]
        bbd = _dot(onehot, bb_ref[...])
        scd = _dot(onehot, sc_ref[...])
        bbsc_ref[...] = jnp.concatenate(
            [bbd, scd, jnp.zeros((s.shape[0], 3), jnp.float32)], axis=1)

    return pl.pallas_call(
        body,
        grid=grid,
        in_specs=[
            pl.BlockSpec((bm, 1), lambda i: (i, 0)),
            pl.BlockSpec((bm, N_ATOM_BASIS), lambda i: (i, 0)),
            pl.BlockSpec((25, RES_DIM), lambda i: (0, 0)),
            pl.BlockSpec((25, 3), lambda i: (0, 0)),
            pl.BlockSpec((25, 10), lambda i: (0, 0)),
            pl.BlockSpec((FP, FP), lambda i: (0, 0)),
            pl.BlockSpec((1, FP), lambda i: (0, 0)),
            pl.BlockSpec((FP, FP), lambda i: (0, 0)),
            pl.BlockSpec((1, FP), lambda i: (0, 0)),
        ],
        out_specs=[
            pl.BlockSpec((bm, FP), lambda i: (i, 0)),
            pl.BlockSpec((bm, HF), lambda i: (i, 0)),
            pl.BlockSpec((bm, HF), lambda i: (i, 0)),
            pl.BlockSpec((bm, 16), lambda i: (i, 0)),
        ],
        out_shape=[
            jax.ShapeDtypeStruct((n_nodes, FP), jnp.float32),
            jax.ShapeDtypeStruct((n_nodes, HF), jnp.float32),
            jax.ShapeDtypeStruct((n_nodes, HF), jnp.float32),
            jax.ShapeDtypeStruct((n_nodes, 16), jnp.float32),
        ],
    )


def _tc_update(n_nodes, with_phi, bm=1000):
    grid = (n_nodes // bm,)

    def body(a0_ref, a1_ref, s_ref, wd1_ref, bd1_ref, wd2_ref, bd2_ref,
             *rest):
        if with_phi:
            (w1_ref, b1_ref, w2_ref, b2_ref,
             sn_ref, phia_ref, phib_ref) = rest
        else:
            (sn_ref,) = rest
        v = jnp.concatenate([a0_ref[...], a1_ref[...]], axis=1)
        t = _silu(v)
        h = _silu(_dot(t, wd1_ref[...]) + bd1_ref[...])
        snext = s_ref[...] + _dot(h, wd2_ref[...]) + bd2_ref[...]
        sn_ref[...] = snext
        if with_phi:
            g = _silu(_dot(snext, w1_ref[...]) + b1_ref[...])
            phi = _dot(g, w2_ref[...]) + b2_ref[...]
            phia_ref[...] = phi[:, :HF]
            phib_ref[...] = phi[:, HF:]

    nmat = pl.BlockSpec((bm, FP), lambda i: (i, 0))
    nhalf = pl.BlockSpec((bm, HF), lambda i: (i, 0))
    wmat = pl.BlockSpec((FP, FP), lambda i: (0, 0))
    wvec = pl.BlockSpec((1, FP), lambda i: (0, 0))
    in_specs = [nhalf, nhalf, nmat, wmat, wvec, wmat, wvec]
    out_specs = [nmat]
    out_shape = [jax.ShapeDtypeStruct((n_nodes, FP), jnp.float32)]
    if with_phi:
        in_specs += [wmat, wvec, wmat, wvec]
        out_specs += [nhalf, nhalf]
        out_shape += [jax.ShapeDtypeStruct((n_nodes, HF), jnp.float32)] * 2
    return pl.pallas_call(body, grid=grid, in_specs=in_specs,
                          out_specs=out_specs, out_shape=out_shape)


def _tc_heads(n_nodes, bm=1000):
    grid = (n_nodes // bm,)

    def body(s_ref, *w_refs):
        ws = [r[...] for r in w_refs[:-1]]
        out_ref = w_refs[-1]
        (ba0w, ba0b, ba1w, ba1b,
         bt0w, bt0b, bt1w, bt1b,
         sa0w, sa0b, sa1w, sa1b,
         t0l1w, t0l1b, t0l2w, t0l2b,
         t1l1w, t1l1b, t1l2w, t1l2b,
         t2l1w, t2l1b, t2l2w, t2l2b,
         ft0w, ft0b, ft1w, ft1b) = ws
        s = s_ref[...]
        a_s = _silu(s)
        bba = _dot(_silu(_dot(a_s, ba0w) + ba0b), ba1w) + ba1b
        t = (_dot(a_s, bt0w[:FP]) + _dot(_silu(bba), bt0w[FP:FP + 3])
             + bt0b)
        bbt = _dot(_silu(t), bt1w) + bt1b
        sca = _dot(_silu(_dot(a_s, sa0w) + sa0b), sa1w) + sa1b
        scs = jnp.concatenate(
            [s, sca, jnp.zeros((s.shape[0], 6), jnp.float32)], axis=1)
        for l1w, l1b, l2w, l2b in ((t0l1w, t0l1b, t0l2w, t0l2b),
                                   (t1l1w, t1l1b, t1l2w, t1l2b),
                                   (t2l1w, t2l1b, t2l2w, t2l2b)):
            h = _silu(_dot(_silu(scs), l1w) + l1b)
            scs = scs + _dot(h, l2w) + l2b
        sct = _dot(_silu(_dot(_silu(scs), ft0w) + ft0b), ft1w) + ft1b
        out_ref[...] = jnp.concatenate([bba, bbt, sca, sct], axis=1)

    nmat = pl.BlockSpec((bm, FP), lambda i: (i, 0))

    def wspec(shape):
        return pl.BlockSpec(shape, lambda i: tuple(0 for _ in shape))

    SP = FP + 16
    in_specs = [nmat]
    wshapes = [
        (FP, 3), (1, 3), (3, 3), (1, 3),
        (FP + 3, 3), (1, 3), (3, 3), (1, 3),
        (FP, 10), (1, 10), (10, 10), (1, 10),
        (SP, SP), (1, SP), (SP, SP), (1, SP),
        (SP, SP), (1, SP), (SP, SP), (1, SP),
        (SP, SP), (1, SP), (SP, SP), (1, SP),
        (SP, 10), (1, 10), (10, 10), (1, 10),
    ]
    in_specs += [wspec(s) for s in wshapes]
    return pl.pallas_call(
        body,
        grid=grid,
        in_specs=in_specs,
        out_specs=pl.BlockSpec((bm, 26), lambda i: (i, 0)),
        out_shape=jax.ShapeDtypeStruct((n_nodes, 26), jnp.float32),
    )


def kernel(cg_z, cg_xyz, CG_nbr_list, mapping, S, params):
    del mapping
    n = S.shape[0]
    e = CG_nbr_list.shape[0]

    src = CG_nbr_list[:, 1].astype(jnp.int32)
    dst = CG_nbr_list[:, 0].astype(jnp.int32)
    xyzT = cg_xyz.T

    d2 = _sc_dist(n, e)(xyzT[0], xyzT[1], xyzT[2], src, dst)

    wdext = []
    for i in range(NUM_CONV):
        mp = params["msg"][i]["dist"]
        wdext.append(jnp.concatenate(
            [_pad2(mp["W"], N_RBF, FP), _pad1(mp["b"], FP)[None, :]], axis=0))
    ws = _tc_prep(e)(d2[:, None], *wdext)

    mp0 = params["msg"][0]
    sfull, phia, phib, bbsc = _tc_embed(n)(
        cg_z.astype(jnp.int32)[:, None], S,
        params["res_embed"], params["backbone_dist"], params["sidechain_dist"],
        _pad2(mp0["inv1"]["W"], FP, FP), _pad1(mp0["inv1"]["b"], FP)[None, :],
        _pad2(mp0["inv2"]["W"], FP, FP), _pad1(mp0["inv2"]["b"], FP)[None, :],
    )

    m = e // NSUB
    c = 0
    for cand in range(128, 0, -8):
        if m % cand == 0:
            c = cand
            break
    src3 = src.reshape(NSUB, m // c, c)
    dst3 = dst.reshape(NSUB, m // c, c)
    n_pad = ((n + NSUB * 8 - 1) // (NSUB * 8)) * (NSUB * 8)
    zeros_tile = jnp.zeros((n_pad // NSUB, HF), jnp.float32)
    sc_conv = _sc_conv(n_pad, e)
    upd = [_tc_update(n, with_phi=True), _tc_update(n, with_phi=True),
           _tc_update(n, with_phi=False)]

    for i in range(NUM_CONV):
        acc0, acc1 = sc_conv(phia, phib, ws[2 * i], ws[2 * i + 1],
                             src3, dst3, zeros_tile)
        acc0 = acc0[:n]
        acc1 = acc1[:n]
        dp = params["dense"][i]
        args = [acc0, acc1, sfull,
                _pad2(dp["l1"]["W"], FP, FP), _pad1(dp["l1"]["b"], FP)[None, :],
                _pad2(dp["l2"]["W"], FP, FP), _pad1(dp["l2"]["b"], FP)[None, :]]
        if i < NUM_CONV - 1:
            mpn = params["msg"][i + 1]
            args += [_pad2(mpn["inv1"]["W"], FP, FP),
                     _pad1(mpn["inv1"]["b"], FP)[None, :],
                     _pad2(mpn["inv2"]["W"], FP, FP),
                     _pad1(mpn["inv2"]["b"], FP)[None, :]]
            sfull, phia, phib = upd[i](*args)
        else:
            (sfull,) = upd[i](*args)

    SP = FP + 16
    p = params

    def tor_pad(w):
        out = jnp.zeros((SP, SP), jnp.float32)
        out = out.at[:FEAT, :FEAT].set(w[:FEAT, :FEAT])
        out = out.at[:FEAT, FP:FP + 10].set(w[:FEAT, FEAT:])
        out = out.at[FP:FP + 10, :FEAT].set(w[FEAT:, :FEAT])
        out = out.at[FP:FP + 10, FP:FP + 10].set(w[FEAT:, FEAT:])
        return out

    def tor_bias(b):
        out = jnp.zeros((SP,), jnp.float32)
        out = out.at[:FEAT].set(b[:FEAT])
        out = out.at[FP:FP + 10].set(b[FEAT:])
        return out

    def ft_pad(w):
        out = jnp.zeros((SP, 10), jnp.float32)
        out = out.at[:FEAT].set(w[:FEAT])
        out = out.at[FP:FP + 10].set(w[FEAT:])
        return out

    wvals = [
        _pad2(p["bb_angle"][0]["W"], FP, 3), p["bb_angle"][0]["b"][None, :],
        p["bb_angle"][1]["W"], p["bb_angle"][1]["b"][None, :],
        _pad2(p["bb_torsion"][0]["W"], FP + 3, 3)
            .at[FP:FP + 3].set(p["bb_torsion"][0]["W"][FEAT:]),
        p["bb_torsion"][0]["b"][None, :],
        p["bb_torsion"][1]["W"], p["bb_torsion"][1]["b"][None, :],
        _pad2(p["sc_angle"][0]["W"], FP, 10), p["sc_angle"][0]["b"][None, :],
        p["sc_angle"][1]["W"], p["sc_angle"][1]["b"][None, :],
    ]
    for i in range(NUM_CONV):
        tp = p["sc_tor"][i]
        wvals += [tor_pad(tp["l1"]["W"]), tor_bias(tp["l1"]["b"])[None, :],
                  tor_pad(tp["l2"]["W"]), tor_bias(tp["l2"]["b"])[None, :]]
    wvals += [ft_pad(p["final_torsion"][0]["W"]),
              p["final_torsion"][0]["b"][None, :],
              p["final_torsion"][1]["W"], p["final_torsion"][1]["b"][None, :]]
    heads = _tc_heads(n)(sfull, *wvals)

    bba = heads[:, 0:3]
    bbt = heads[:, 3:6]
    sca = heads[:, 6:16]
    sct = heads[:, 16:26]
    bbd = bbsc[:, 0:3]
    scd = bbsc[:, 3:13]
    ic_bb = jnp.stack([bbd, bba, bbt], axis=-1)
    ic_sc = jnp.stack([scd, sca, sct], axis=-1)
    return jnp.concatenate([ic_bb, ic_sc], axis=-2)


# speedup vs baseline: 3.0595x; 1.1753x over previous
import functools

import jax
import jax.numpy as jnp
import numpy as np
from jax import lax
from jax.experimental import pallas as pl
from jax.experimental.pallas import tpu as pltpu
from jax.experimental.pallas import tpu_sc as plsc

N_ATOM_BASIS = 128
RES_DIM = 4
FEAT = N_ATOM_BASIS + RES_DIM
FP = 160
HF = FP // 2
N_RBF = 15
CUTOFF = 21.0
NUM_CONV = 3

NSUB = 16
NCORE = 2
NW = NCORE * NSUB

_SC_PARAMS = pltpu.CompilerParams(needs_layout_passes=False,
                                  use_tc_tiling_on_sc=False)


def _silu(x):
    return x * jax.nn.sigmoid(x)


def _pad2(w, rows, cols):
    return jnp.zeros((rows, cols), w.dtype).at[: w.shape[0], : w.shape[1]].set(w)


def _pad1(b, n):
    return jnp.zeros((n,), b.dtype).at[: b.shape[0]].set(b)


def _sc_dist(n_nodes, n_edges):
    m = n_edges // NW
    assert n_edges % NW == 0 and m % 16 == 0

    mesh = plsc.VectorSubcoreMesh(core_axis_name="c", subcore_axis_name="s",
                                  num_cores=NCORE, num_subcores=NSUB)

    @functools.partial(
        pl.kernel,
        out_type=jax.ShapeDtypeStruct((n_edges,), jnp.float32),
        mesh=mesh,
        scratch_types=[
            pltpu.VMEM((n_nodes,), jnp.float32),
            pltpu.VMEM((n_nodes,), jnp.float32),
            pltpu.VMEM((n_nodes,), jnp.float32),
            pltpu.VMEM((m,), jnp.int32),
            pltpu.VMEM((m,), jnp.int32),
            pltpu.VMEM((m,), jnp.float32),
        ],
        compiler_params=_SC_PARAMS,
    )
    def k(x_hbm, y_hbm, z_hbm, src_hbm, dst_hbm, out_hbm, xv, yv, zv,
          sv, dv, ov):
        cid = lax.axis_index("c")
        sid = lax.axis_index("s")
        wid = cid * NSUB + sid
        base = wid * m
        pltpu.sync_copy(x_hbm, xv)
        pltpu.sync_copy(y_hbm, yv)
        pltpu.sync_copy(z_hbm, zv)
        pltpu.sync_copy(src_hbm.at[pl.ds(base, m)], sv)
        pltpu.sync_copy(dst_hbm.at[pl.ds(base, m)], dv)

        def body(i, _):
            si = sv[pl.ds(i * 16, 16)]
            di = dv[pl.ds(i * 16, 16)]
            dx = plsc.load_gather(xv, [si]) - plsc.load_gather(xv, [di])
            dy = plsc.load_gather(yv, [si]) - plsc.load_gather(yv, [di])
            dz = plsc.load_gather(zv, [si]) - plsc.load_gather(zv, [di])
            ov[pl.ds(i * 16, 16)] = dx * dx + dy * dy + dz * dz
            return ()

        lax.fori_loop(0, m // 16, body, ())
        pltpu.sync_copy(ov, out_hbm.at[pl.ds(base, m)])

    return k


def _sc_conv(n_pad, n_edges):
    m = n_edges // NSUB
    c = 0
    for cand in range(128, 0, -8):
        if m % cand == 0:
            c = cand
            break
    assert c > 0
    j_chunks = m // c
    rows_pt = n_pad // NSUB
    assert n_pad % (NSUB * 8) == 0

    mesh = plsc.VectorSubcoreMesh(core_axis_name="c", subcore_axis_name="s",
                                  num_cores=NCORE, num_subcores=NSUB)

    @functools.partial(
        pl.kernel,
        out_type=[
            jax.ShapeDtypeStruct((n_pad, HF), jnp.float32),
            jax.ShapeDtypeStruct((n_pad, HF), jnp.float32),
        ],
        mesh=mesh,
        scratch_types=[
            pltpu.VMEM((j_chunks, c), jnp.int32),
            pltpu.VMEM((j_chunks, c), jnp.int32),
            pltpu.VMEM((2, c, HF), jnp.float32),
            pltpu.VMEM((2, c, HF), jnp.float32),
            pltpu.VMEM((2, c, HF), jnp.float32),
            pltpu.VMEM_SHARED((n_pad, HF), jnp.float32),
            pltpu.SemaphoreType.DMA,
            pltpu.SemaphoreType.DMA,
            pltpu.SemaphoreType.DMA,
            pltpu.SemaphoreType.DMA,
            pltpu.SemaphoreType.DMA,
            pltpu.SemaphoreType.DMA,
        ],
        compiler_params=_SC_PARAMS,
    )
    def k(phia_hbm, phib_hbm, wsa_hbm, wsb_hbm, src_hbm, dst_hbm, zero_hbm,
          out0, out1, sv, dv, pb, wb, qb, acc,
          sg0, sg1, sw0, sw1, ss0, ss1):
        cid = lax.axis_index("c")
        sid = lax.axis_index("s")
        pltpu.sync_copy(zero_hbm, acc.at[pl.ds(sid * rows_pt, rows_pt)])
        pltpu.sync_copy(src_hbm.at[sid], sv)
        pltpu.sync_copy(dst_hbm.at[sid], dv)
        plsc.subcore_barrier()
        sgs = (sg0, sg1)
        sws = (sw0, sw1)
        sss = (ss0, ss1)

        def run(phi_hbm, ws_hbm, out_hbm):
            def issue(j, b):
                pltpu.async_copy(phi_hbm.at[sv.at[j]], pb.at[b], sgs[b])
                pltpu.async_copy(ws_hbm.at[pl.ds(sid * m + j * c, c)],
                                 wb.at[b], sws[b])

            issue(0, 0)

            def pair(jj, _):
                for b in (0, 1):
                    j = jj * 2 + b
                    b1 = 1 - b

                    @pl.when(j + 1 < j_chunks)
                    def _():
                        issue(j + 1, b1)

                    pltpu.make_async_copy(phi_hbm.at[sv.at[j]], pb.at[b],
                                          sgs[b]).wait()
                    pltpu.make_async_copy(ws_hbm.at[pl.ds(0, c)], wb.at[b],
                                          sws[b]).wait()

                    @pl.when(jj >= 1)
                    def _():
                        pltpu.make_async_copy(qb.at[b], acc.at[dv.at[j]],
                                              sss[b]).wait()

                    def rowmul(r, _):
                        for rr in (0, 1):
                            for q in range(HF // 16):
                                s = pl.ds(q * 16, 16)
                                qb[b, 2 * r + rr, s] = (pb[b, 2 * r + rr, s]
                                                        * wb[b, 2 * r + rr, s])
                        return ()

                    lax.fori_loop(0, c // 2, rowmul, ())
                    pltpu.async_copy(qb.at[b], acc.at[dv.at[j]], sss[b],
                                     add=True)
                return ()

            lax.fori_loop(0, j_chunks // 2, pair, ())
            pltpu.make_async_copy(qb.at[0], acc.at[dv.at[0]], sss[0]).wait()
            pltpu.make_async_copy(qb.at[1], acc.at[dv.at[1]], sss[1]).wait()
            plsc.subcore_barrier()
            row0 = sid * rows_pt
            pltpu.sync_copy(acc.at[pl.ds(row0, rows_pt)],
                            out_hbm.at[pl.ds(row0, rows_pt)])

        @pl.when(cid == 0)
        def _():
            run(phia_hbm, wsa_hbm, out0)

        @pl.when(cid == 1)
        def _():
            run(phib_hbm, wsb_hbm, out1)

    return k


def _dot(a, b):
    return jax.lax.dot_general(a, b, (((1,), (0,)), ((), ())),
                               preferred_element_type=jnp.float32)


def _tc_prep(n_edges, be=4000):
    grid = (n_edges // be,)
    mu0 = float(np.exp(-CUTOFF))
    mu_step = (1.0 - mu0) / (N_RBF - 1)
    beta = float((2.0 / N_RBF * (1.0 - np.exp(-CUTOFF))) ** -2)

    def body(d2_ref, w1_ref, w2_ref, w3_ref, *o_refs):
        d2 = d2_ref[...]
        dist = jnp.sqrt(d2)
        env = jnp.where(dist < CUTOFF,
                        0.5 * (jnp.cos(dist * (np.pi / CUTOFF)) + 1.0),
                        0.0)
        ex = jnp.exp(-dist)
        mu = (mu0 + mu_step * jax.lax.broadcasted_iota(
            jnp.int32, (1, N_RBF), 1).astype(jnp.float32))
        rbf = jnp.exp(-beta * (ex - mu) ** 2)
        rext = jnp.concatenate([rbf, jnp.ones_like(d2)], axis=1) * env
        for i, w_ref in enumerate((w1_ref, w2_ref, w3_ref)):
            ws = _dot(rext, w_ref[...])
            o_refs[2 * i][...] = ws[:, :HF]
            o_refs[2 * i + 1][...] = ws[:, HF:]

    half = jax.ShapeDtypeStruct((n_edges, HF), jnp.float32)
    return pl.pallas_call(
        body,
        grid=grid,
        in_specs=[
            pl.BlockSpec((be, 1), lambda i: (i, 0)),
            pl.BlockSpec((16, FP), lambda i: (0, 0)),
            pl.BlockSpec((16, FP), lambda i: (0, 0)),
            pl.BlockSpec((16, FP), lambda i: (0, 0)),
        ],
        out_specs=[pl.BlockSpec((be, HF), lambda i: (i, 0))] * 6,
        out_shape=[half] * 6,
    )


def _tc_embed(n_nodes, bm=1000):
    grid = (n_nodes // bm,)

    def body(z_ref, s_ref, re_ref, bb_ref, sc_ref, w1_ref, b1_ref,
             w2_ref, b2_ref, sf_ref, phia_ref, phib_ref, bbsc_ref):
        z = z_ref[...]
        onehot = (z == jax.lax.broadcasted_iota(jnp.int32, (1, 25), 1)
                  ).astype(jnp.float32)
        res4 = _dot(onehot, re_ref[...])
        s = s_ref[...]
        pad = jnp.zeros((s.shape[0], FP - FEAT), jnp.float32)
        sfull = jnp.concatenate([s, res4, pad], axis=1)
        sf_ref[...] = sfull
        h = _silu(_dot(sfull, w1_ref[...]) + b1_ref[...])
        phi = _dot(h, w2_ref[...]) + b2_ref[...]
        phia_ref[...] = phi[:, :HF]
        phib_ref[...] = phi[:, HF:]
        bbd = _dot(onehot, bb_ref[...])
        scd = _dot(onehot, sc_ref[...])
        bbsc_ref[...] = jnp.concatenate(
            [bbd, scd, jnp.zeros((s.shape[0], 3), jnp.float32)], axis=1)

    return pl.pallas_call(
        body,
        grid=grid,
        in_specs=[
            pl.BlockSpec((bm, 1), lambda i: (i, 0)),
            pl.BlockSpec((bm, N_ATOM_BASIS), lambda i: (i, 0)),
            pl.BlockSpec((25, RES_DIM), lambda i: (0, 0)),
            pl.BlockSpec((25, 3), lambda i: (0, 0)),
            pl.BlockSpec((25, 10), lambda i: (0, 0)),
            pl.BlockSpec((FP, FP), lambda i: (0, 0)),
            pl.BlockSpec((1, FP), lambda i: (0, 0)),
            pl.BlockSpec((FP, FP), lambda i: (0, 0)),
            pl.BlockSpec((1, FP), lambda i: (0, 0)),
        ],
        out_specs=[
            pl.BlockSpec((bm, FP), lambda i: (i, 0)),
            pl.BlockSpec((bm, HF), lambda i: (i, 0)),
            pl.BlockSpec((bm, HF), lambda i: (i, 0)),
            pl.BlockSpec((bm, 16), lambda i: (i, 0)),
        ],
        out_shape=[
            jax.ShapeDtypeStruct((n_nodes, FP), jnp.float32),
            jax.ShapeDtypeStruct((n_nodes, HF), jnp.float32),
            jax.ShapeDtypeStruct((n_nodes, HF), jnp.float32),
            jax.ShapeDtypeStruct((n_nodes, 16), jnp.float32),
        ],
    )


def _tc_update(n_nodes, with_phi, bm=1000):
    grid = (n_nodes // bm,)

    def body(a0_ref, a1_ref, s_ref, wd1_ref, bd1_ref, wd2_ref, bd2_ref,
             *rest):
        if with_phi:
            (w1_ref, b1_ref, w2_ref, b2_ref,
             sn_ref, phia_ref, phib_ref) = rest
        else:
            (sn_ref,) = rest
        v = jnp.concatenate([a0_ref[...], a1_ref[...]], axis=1)
        t = _silu(v)
        h = _silu(_dot(t, wd1_ref[...]) + bd1_ref[...])
        snext = s_ref[...] + _dot(h, wd2_ref[...]) + bd2_ref[...]
        sn_ref[...] = snext
        if with_phi:
            g = _silu(_dot(snext, w1_ref[...]) + b1_ref[...])
            phi = _dot(g, w2_ref[...]) + b2_ref[...]
            phia_ref[...] = phi[:, :HF]
            phib_ref[...] = phi[:, HF:]

    nmat = pl.BlockSpec((bm, FP), lambda i: (i, 0))
    nhalf = pl.BlockSpec((bm, HF), lambda i: (i, 0))
    wmat = pl.BlockSpec((FP, FP), lambda i: (0, 0))
    wvec = pl.BlockSpec((1, FP), lambda i: (0, 0))
    in_specs = [nhalf, nhalf, nmat, wmat, wvec, wmat, wvec]
    out_specs = [nmat]
    out_shape = [jax.ShapeDtypeStruct((n_nodes, FP), jnp.float32)]
    if with_phi:
        in_specs += [wmat, wvec, wmat, wvec]
        out_specs += [nhalf, nhalf]
        out_shape += [jax.ShapeDtypeStruct((n_nodes, HF), jnp.float32)] * 2
    return pl.pallas_call(body, grid=grid, in_specs=in_specs,
                          out_specs=out_specs, out_shape=out_shape)


def _tc_heads(n_nodes, bm=1000):
    grid = (n_nodes // bm,)

    def body(s_ref, *w_refs):
        ws = [r[...] for r in w_refs[:-1]]
        out_ref = w_refs[-1]
        (ba0w, ba0b, ba1w, ba1b,
         bt0w, bt0b, bt1w, bt1b,
         sa0w, sa0b, sa1w, sa1b,
         t0l1w, t0l1b, t0l2w, t0l2b,
         t1l1w, t1l1b, t1l2w, t1l2b,
         t2l1w, t2l1b, t2l2w, t2l2b,
         ft0w, ft0b, ft1w, ft1b) = ws
        s = s_ref[...]
        a_s = _silu(s)
        bba = _dot(_silu(_dot(a_s, ba0w) + ba0b), ba1w) + ba1b
        t = (_dot(a_s, bt0w[:FP]) + _dot(_silu(bba), bt0w[FP:FP + 3])
             + bt0b)
        bbt = _dot(_silu(t), bt1w) + bt1b
        sca = _dot(_silu(_dot(a_s, sa0w) + sa0b), sa1w) + sa1b
        scs = jnp.concatenate(
            [s, sca, jnp.zeros((s.shape[0], 6), jnp.float32)], axis=1)
        for l1w, l1b, l2w, l2b in ((t0l1w, t0l1b, t0l2w, t0l2b),
                                   (t1l1w, t1l1b, t1l2w, t1l2b),
                                   (t2l1w, t2l1b, t2l2w, t2l2b)):
            h = _silu(_dot(_silu(scs), l1w) + l1b)
            scs = scs + _dot(h, l2w) + l2b
        sct = _dot(_silu(_dot(_silu(scs), ft0w) + ft0b), ft1w) + ft1b
        out_ref[...] = jnp.concatenate([bba, bbt, sca, sct], axis=1)

    nmat = pl.BlockSpec((bm, FP), lambda i: (i, 0))

    def wspec(shape):
        return pl.BlockSpec(shape, lambda i: tuple(0 for _ in shape))

    SP = FP + 16
    in_specs = [nmat]
    wshapes = [
        (FP, 3), (1, 3), (3, 3), (1, 3),
        (FP + 3, 3), (1, 3), (3, 3), (1, 3),
        (FP, 10), (1, 10), (10, 10), (1, 10),
        (SP, SP), (1, SP), (SP, SP), (1, SP),
        (SP, SP), (1, SP), (SP, SP), (1, SP),
        (SP, SP), (1, SP), (SP, SP), (1, SP),
        (SP, 10), (1, 10), (10, 10), (1, 10),
    ]
    in_specs += [wspec(s) for s in wshapes]
    return pl.pallas_call(
        body,
        grid=grid,
        in_specs=in_specs,
        out_specs=pl.BlockSpec((bm, 26), lambda i: (i, 0)),
        out_shape=jax.ShapeDtypeStruct((n_nodes, 26), jnp.float32),
    )


def kernel(cg_z, cg_xyz, CG_nbr_list, mapping, S, params):
    del mapping
    n = S.shape[0]
    e = CG_nbr_list.shape[0]

    src = CG_nbr_list[:, 1].astype(jnp.int32)
    dst = CG_nbr_list[:, 0].astype(jnp.int32)
    xyzT = cg_xyz.T

    d2 = _sc_dist(n, e)(xyzT[0], xyzT[1], xyzT[2], src, dst)

    wdext = []
    for i in range(NUM_CONV):
        mp = params["msg"][i]["dist"]
        wdext.append(jnp.concatenate(
            [_pad2(mp["W"], N_RBF, FP), _pad1(mp["b"], FP)[None, :]], axis=0))
    ws = _tc_prep(e)(d2[:, None], *wdext)

    mp0 = params["msg"][0]
    sfull, phia, phib, bbsc = _tc_embed(n)(
        cg_z.astype(jnp.int32)[:, None], S,
        params["res_embed"], params["backbone_dist"], params["sidechain_dist"],
        _pad2(mp0["inv1"]["W"], FP, FP), _pad1(mp0["inv1"]["b"], FP)[None, :],
        _pad2(mp0["inv2"]["W"], FP, FP), _pad1(mp0["inv2"]["b"], FP)[None, :],
    )

    m = e // NSUB
    c = 0
    for cand in range(128, 0, -8):
        if m % cand == 0:
            c = cand
            break
    src3 = src.reshape(NSUB, m // c, c)
    dst3 = dst.reshape(NSUB, m // c, c)
    n_pad = ((n + NSUB * 8 - 1) // (NSUB * 8)) * (NSUB * 8)
    zeros_tile = jnp.zeros((n_pad // NSUB, HF), jnp.float32)
    sc_conv = _sc_conv(n_pad, e)
    upd = [_tc_update(n, with_phi=True), _tc_update(n, with_phi=True),
           _tc_update(n, with_phi=False)]

    for i in range(NUM_CONV):
        acc0, acc1 = sc_conv(phia, phib, ws[2 * i], ws[2 * i + 1],
                             src3, dst3, zeros_tile)
        acc0 = acc0[:n]
        acc1 = acc1[:n]
        dp = params["dense"][i]
        args = [acc0, acc1, sfull,
                _pad2(dp["l1"]["W"], FP, FP), _pad1(dp["l1"]["b"], FP)[None, :],
                _pad2(dp["l2"]["W"], FP, FP), _pad1(dp["l2"]["b"], FP)[None, :]]
        if i < NUM_CONV - 1:
            mpn = params["msg"][i + 1]
            args += [_pad2(mpn["inv1"]["W"], FP, FP),
                     _pad1(mpn["inv1"]["b"], FP)[None, :],
                     _pad2(mpn["inv2"]["W"], FP, FP),
                     _pad1(mpn["inv2"]["b"], FP)[None, :]]
            sfull, phia, phib = upd[i](*args)
        else:
            (sfull,) = upd[i](*args)

    SP = FP + 16
    p = params

    def tor_pad(w):
        out = jnp.zeros((SP, SP), jnp.float32)
        out = out.at[:FEAT, :FEAT].set(w[:FEAT, :FEAT])
        out = out.at[:FEAT, FP:FP + 10].set(w[:FEAT, FEAT:])
        out = out.at[FP:FP + 10, :FEAT].set(w[FEAT:, :FEAT])
        out = out.at[FP:FP + 10, FP:FP + 10].set(w[FEAT:, FEAT:])
        return out

    def tor_bias(b):
        out = jnp.zeros((SP,), jnp.float32)
        out = out.at[:FEAT].set(b[:FEAT])
        out = out.at[FP:FP + 10].set(b[FEAT:])
        return out

    def ft_pad(w):
        out = jnp.zeros((SP, 10), jnp.float32)
        out = out.at[:FEAT].set(w[:FEAT])
        out = out.at[FP:FP + 10].set(w[FEAT:])
        return out

    wvals = [
        _pad2(p["bb_angle"][0]["W"], FP, 3), p["bb_angle"][0]["b"][None, :],
        p["bb_angle"][1]["W"], p["bb_angle"][1]["b"][None, :],
        _pad2(p["bb_torsion"][0]["W"], FP + 3, 3)
            .at[FP:FP + 3].set(p["bb_torsion"][0]["W"][FEAT:]),
        p["bb_torsion"][0]["b"][None, :],
        p["bb_torsion"][1]["W"], p["bb_torsion"][1]["b"][None, :],
        _pad2(p["sc_angle"][0]["W"], FP, 10), p["sc_angle"][0]["b"][None, :],
        p["sc_angle"][1]["W"], p["sc_angle"][1]["b"][None, :],
    ]
    for i in range(NUM_CONV):
        tp = p["sc_tor"][i]
        wvals += [tor_pad(tp["l1"]["W"]), tor_bias(tp["l1"]["b"])[None, :],
                  tor_pad(tp["l2"]["W"]), tor_bias(tp["l2"]["b"])[None, :]]
    wvals += [ft_pad(p["final_torsion"][0]["W"]),
              p["final_torsion"][0]["b"][None, :],
              p["final_torsion"][1]["W"], p["final_torsion"][1]["b"][None, :]]
    heads = _tc_heads(n)(sfull, *wvals)

    bba = heads[:, 0:3]
    bbt = heads[:, 3:6]
    sca = heads[:, 6:16]
    sct = heads[:, 16:26]
    bbd = bbsc[:, 0:3]
    scd = bbsc[:, 3:13]
    ic_bb = jnp.stack([bbd, bba, bbt], axis=-1)
    ic_sc = jnp.stack([scd, sca, sct], axis=-1)
    return jnp.concatenate([ic_bb, ic_sc], axis=-2)
